# baseline (speedup 1.0000x reference)
import functools

import jax
import jax.numpy as jnp
from jax import lax
from jax.experimental import pallas as pl
from jax.experimental.pallas import tpu as pltpu
from jax.experimental.pallas import tpu_sc as plsc

N = 10000
E = 320000
H1 = 32
H2 = 32

NS = 16
EPT = E // NS

DST_OFF = 10240
NPAD = 2 * DST_OFF

CH = 128
NCH = 160
EPAD = NS * NCH * CH
NBUF = 5
RPT = 640
NROW = NS * RPT

_f32 = jnp.float32


def _rsqrt16(d):
    magic = jnp.full((16,), 0x5F3759DF, jnp.int32)
    y = plsc.bitcast(magic - (plsc.bitcast(d, jnp.int32) >> 1), _f32)
    for _ in range(3):
        y = y * (1.5 - 0.5 * d * y * y)
    return y


def _tc_z1_body(x_ref, w_ref, b_ref, z1_ref):
    z1_ref[0:N] = (
        jnp.dot(x_ref[...], w_ref[...], preferred_element_type=_f32)
        + b_ref[...]
    )
    z1_ref[N:NROW] = jnp.zeros((NROW - N, H1), _f32)


def _tc_z1(x, w_lin, b_lin):
    return pl.pallas_call(
        _tc_z1_body,
        out_shape=jax.ShapeDtypeStruct((NROW, H1), _f32),
    )(x, w_lin, b_lin)


def _sc_norms_body(src_hbm, dst_hbm, norm_hbm,
                   srcids_v, dstids_v, hist_v, slab_v,
                   normsrc_v, normdst_v, hstage_sh):
    sub = lax.axis_index("s")
    rowbase = sub * RPT

    pltpu.sync_copy(src_hbm.at[pl.ds(sub * EPT, EPT)], srcids_v)
    pltpu.sync_copy(dst_hbm.at[pl.ds(sub * EPT, EPT)], dstids_v)

    zeros16 = jnp.zeros((16,), _f32)
    ones16 = jnp.ones((16,), _f32)

    def _zero_hist(i, _):
        for u in range(8):
            hist_v[pl.ds((i * 8 + u) * 16, 16)] = zeros16
        return _

    lax.fori_loop(0, NPAD // 128, _zero_hist, None)

    def _hist(r, _):
        for c in range(5):
            s = srcids_v[pl.ds((r * 5 + c) * 16, 16)]
            plsc.addupdate_scatter(hist_v, [s], ones16)
            d = dstids_v[pl.ds((r * 5 + c) * 16, 16)] + DST_OFF
            plsc.addupdate_scatter(hist_v, [d], ones16)
        return _

    lax.fori_loop(0, EPT // 80, _hist, None)

    pltpu.sync_copy(hist_v, hstage_sh.at[sub])
    plsc.subcore_barrier()

    def _reduce_norm(col0, norm_v):
        pltpu.sync_copy(hstage_sh.at[:, pl.ds(col0, RPT)], slab_v)

        def _red(c, _):
            acc = zeros16
            for r in range(NS):
                acc = acc + slab_v[r, pl.ds(c * 16, 16)]
            norm_v[pl.ds(c * 16, 16)] = _rsqrt16(jnp.maximum(acc, 1.0))
            return _

        lax.fori_loop(0, RPT // 16, _red, None)

    _reduce_norm(rowbase, normsrc_v)
    _reduce_norm(DST_OFF + rowbase, normdst_v)
    pltpu.sync_copy(normsrc_v, norm_hbm.at[pl.ds(rowbase, RPT)])
    pltpu.sync_copy(normdst_v, norm_hbm.at[pl.ds(DST_OFF + rowbase, RPT)])


def _sc_norms(src, dst):
    mesh = plsc.VectorSubcoreMesh(
        core_axis_name="c", subcore_axis_name="s", num_cores=1)
    call = pl.kernel(
        _sc_norms_body,
        out_type=jax.ShapeDtypeStruct((NPAD,), _f32),
        mesh=mesh,
        scratch_types=[
            pltpu.VMEM((EPT,), jnp.int32),
            pltpu.VMEM((EPT,), jnp.int32),
            pltpu.VMEM((NPAD,), _f32),
            pltpu.VMEM((NS, RPT), _f32),
            pltpu.VMEM((RPT,), _f32),
            pltpu.VMEM((RPT,), _f32),
            pltpu.VMEM_SHARED((NS, NPAD), _f32),
        ],
        compiler_params=pltpu.CompilerParams(
            needs_layout_passes=False, use_tc_tiling_on_sc=False
        ),
    )
    return call(src, dst)


def _sc_body(z1_hbm, norm_hbm, src2d_hbm, dst2d_hbm, scaled_hbm, out_hbm,
             srcids_v, dstids_v, zrows_v, rows_v,
             normsrc_v, normdst_v, agg_sh, *sems):
    sub = lax.axis_index("s")
    rowbase = sub * RPT

    pltpu.sync_copy(src2d_hbm.at[pl.ds(sub * NCH, NCH)], srcids_v)
    pltpu.sync_copy(dst2d_hbm.at[pl.ds(sub * NCH, NCH)], dstids_v)
    pltpu.sync_copy(norm_hbm.at[pl.ds(rowbase, RPT)], normsrc_v)
    pltpu.sync_copy(norm_hbm.at[pl.ds(DST_OFF + rowbase, RPT)], normdst_v)

    zeros16 = jnp.zeros((16,), _f32)

    pltpu.sync_copy(z1_hbm.at[pl.ds(rowbase, RPT)], zrows_v)

    def _scale_rows(norm_v):
        def _scale_grp(g, _):
            nv = norm_v[pl.ds(g * 16, 16)]
            for u in range(16):
                s = nv[u]
                r = g * 16 + u
                zrows_v[r, pl.ds(0, 16)] = zrows_v[r, pl.ds(0, 16)] * s
                zrows_v[r, pl.ds(16, 16)] = zrows_v[r, pl.ds(16, 16)] * s
            return _

        lax.fori_loop(0, RPT // 16, _scale_grp, None)

    _scale_rows(normsrc_v)
    pltpu.sync_copy(zrows_v, scaled_hbm.at[pl.ds(rowbase, RPT)])

    def _zero_rows(r, _):
        zrows_v[r, pl.ds(0, 16)] = zeros16
        zrows_v[r, pl.ds(16, 16)] = zeros16
        return _

    lax.fori_loop(0, RPT, _zero_rows, None)
    pltpu.sync_copy(zrows_v, agg_sh.at[pl.ds(rowbase, RPT)])
    plsc.subcore_barrier()

    def _gather_wait(i, b):
        pltpu.make_async_copy(
            scaled_hbm.at[srcids_v.at[i]], rows_v.at[b], sems[b]).wait()

    def _scatter_wait(i, b):
        pltpu.make_async_copy(
            rows_v.at[b], agg_sh.at[dstids_v.at[i]], sems[NBUF + b]).wait()

    for b in range(NBUF):
        pltpu.async_copy(scaled_hbm.at[srcids_v.at[b]], rows_v.at[b], sems[b])

    def _group(g, _):
        for b in range(NBUF):
            i = g * NBUF + b
            _gather_wait(i, b)
            pltpu.async_copy(
                rows_v.at[b], agg_sh.at[dstids_v.at[i]], sems[NBUF + b],
                add=True)

            @pl.when(g < NCH // NBUF - 1)
            def _():
                _scatter_wait(i, b)
                pltpu.async_copy(
                    scaled_hbm.at[srcids_v.at[i + NBUF]], rows_v.at[b],
                    sems[b])

        return _

    lax.fori_loop(0, NCH // NBUF, _group, None)
    for b in range(NBUF):
        _scatter_wait(NCH - NBUF + b, b)

    plsc.subcore_barrier()

    pltpu.sync_copy(agg_sh.at[pl.ds(rowbase, RPT)], zrows_v)
    _scale_rows(normdst_v)
    pltpu.sync_copy(zrows_v, out_hbm.at[pl.ds(rowbase, RPT)])


def _sc_aggregate(z1, norm, src2d, dst2d):
    mesh = plsc.VectorSubcoreMesh(
        core_axis_name="c", subcore_axis_name="s", num_cores=1)
    call = pl.kernel(
        _sc_body,
        out_type=[
            jax.ShapeDtypeStruct((NROW, H1), _f32),
            jax.ShapeDtypeStruct((NROW, H1), _f32),
        ],
        mesh=mesh,
        scratch_types=[
            pltpu.VMEM((NCH, CH), jnp.int32),
            pltpu.VMEM((NCH, CH), jnp.int32),
            pltpu.VMEM((RPT, H1), _f32),
            pltpu.VMEM((NBUF, CH, H1), _f32),
            pltpu.VMEM((RPT,), _f32),
            pltpu.VMEM((RPT,), _f32),
            pltpu.VMEM_SHARED((NROW, H1), _f32),
        ] + [pltpu.SemaphoreType.DMA] * (2 * NBUF),
        compiler_params=pltpu.CompilerParams(
            needs_layout_passes=False, use_tc_tiling_on_sc=False
        ),
    )
    return call(z1, norm, src2d, dst2d)


def _tc_post_body(p_ref, w_ref, b_ref, out_ref):
    z = jnp.dot(p_ref[0:N], w_ref[...], preferred_element_type=_f32)
    out_ref[...] = jnp.maximum(z + b_ref[...], 0.0)


def _tc_post(agg, w_gc, b_gc):
    return pl.pallas_call(
        _tc_post_body,
        out_shape=jax.ShapeDtypeStruct((N, H2), _f32),
    )(agg, w_gc, b_gc)


def kernel(x, edge_index, W_lin, b_lin, W_gc, b_gc):
    src = edge_index[0]
    dst = edge_index[1]
    pad = jnp.full((NS, (EPAD - E) // NS), N + 16, jnp.int32)
    src2d = jnp.concatenate(
        [src.reshape(NS, EPT), pad], axis=1).reshape(NS * NCH, CH)
    dst2d = jnp.concatenate(
        [dst.reshape(NS, EPT), pad], axis=1).reshape(NS * NCH, CH)
    norm = _sc_norms(src, dst)
    z1 = _tc_z1(x, W_lin, b_lin.reshape(1, H1))
    _, agg = _sc_aggregate(z1, norm, src2d, dst2d)
    return _tc_post(agg, W_gc, b_gc.reshape(1, H2))


# speedup vs baseline: 1.5461x; 1.5461x over previous
import functools

import jax
import jax.numpy as jnp
from jax import lax
from jax.experimental import pallas as pl
from jax.experimental.pallas import tpu as pltpu
from jax.experimental.pallas import tpu_sc as plsc

N = 10000
E = 320000
H1 = 32
H2 = 32

NS = 16
EPT = E // NS

DST_OFF = 10240
NPAD = 2 * DST_OFF

CH = 80
NCH = EPT // CH
NBUF = 10
RPT = 640
NROW = NS * RPT

_f32 = jnp.float32


def _rsqrt16(d):
    magic = jnp.full((16,), 0x5F3759DF, jnp.int32)
    y = plsc.bitcast(magic - (plsc.bitcast(d, jnp.int32) >> 1), _f32)
    for _ in range(3):
        y = y * (1.5 - 0.5 * d * y * y)
    return y


def _tc_z1_body(x_ref, w_ref, b_ref, z1_ref):
    z1_ref[0:N] = (
        jnp.dot(x_ref[...], w_ref[...], preferred_element_type=_f32)
        + b_ref[...]
    )
    z1_ref[N:NROW] = jnp.zeros((NROW - N, H1), _f32)


def _tc_z1(x, w_lin, b_lin):
    return pl.pallas_call(
        _tc_z1_body,
        out_shape=jax.ShapeDtypeStruct((NROW, H1), _f32),
    )(x, w_lin, b_lin)


def _sc_norms_body(src_hbm, dst_hbm, norm_hbm,
                   srcids_v, dstids_v, hist_v, slab_v,
                   normsrc_v, normdst_v, hstage_sh):
    sub = lax.axis_index("s")
    rowbase = sub * RPT

    pltpu.sync_copy(src_hbm.at[pl.ds(sub * EPT, EPT)], srcids_v)
    pltpu.sync_copy(dst_hbm.at[pl.ds(sub * EPT, EPT)], dstids_v)

    zeros16 = jnp.zeros((16,), _f32)
    ones16 = jnp.ones((16,), _f32)

    def _zero_hist(i, _):
        for u in range(8):
            hist_v[pl.ds((i * 8 + u) * 16, 16)] = zeros16
        return _

    lax.fori_loop(0, NPAD // 128, _zero_hist, None)

    def _hist(r, _):
        for c in range(5):
            s = srcids_v[pl.ds((r * 5 + c) * 16, 16)]
            plsc.addupdate_scatter(hist_v, [s], ones16)
            d = dstids_v[pl.ds((r * 5 + c) * 16, 16)] + DST_OFF
            plsc.addupdate_scatter(hist_v, [d], ones16)
        return _

    lax.fori_loop(0, EPT // 80, _hist, None)

    pltpu.sync_copy(hist_v, hstage_sh.at[sub])
    plsc.subcore_barrier()

    def _reduce_norm(col0, norm_v):
        pltpu.sync_copy(hstage_sh.at[:, pl.ds(col0, RPT)], slab_v)

        def _red(c, _):
            acc = zeros16
            for r in range(NS):
                acc = acc + slab_v[r, pl.ds(c * 16, 16)]
            norm_v[pl.ds(c * 16, 16)] = _rsqrt16(jnp.maximum(acc, 1.0))
            return _

        lax.fori_loop(0, RPT // 16, _red, None)

    _reduce_norm(rowbase, normsrc_v)
    _reduce_norm(DST_OFF + rowbase, normdst_v)
    pltpu.sync_copy(normsrc_v, norm_hbm.at[pl.ds(rowbase, RPT)])
    pltpu.sync_copy(normdst_v, norm_hbm.at[pl.ds(DST_OFF + rowbase, RPT)])


def _sc_norms(src, dst):
    mesh = plsc.VectorSubcoreMesh(
        core_axis_name="c", subcore_axis_name="s", num_cores=1)
    call = pl.kernel(
        _sc_norms_body,
        out_type=jax.ShapeDtypeStruct((NPAD,), _f32),
        mesh=mesh,
        scratch_types=[
            pltpu.VMEM((EPT,), jnp.int32),
            pltpu.VMEM((EPT,), jnp.int32),
            pltpu.VMEM((NPAD,), _f32),
            pltpu.VMEM((NS, RPT), _f32),
            pltpu.VMEM((RPT,), _f32),
            pltpu.VMEM((RPT,), _f32),
            pltpu.VMEM_SHARED((NS, NPAD), _f32),
        ],
        compiler_params=pltpu.CompilerParams(
            needs_layout_passes=False, use_tc_tiling_on_sc=False
        ),
    )
    return call(src, dst)


def _sc_body(z1_hbm, norm_hbm, src2d_hbm, dst2d_hbm, scaled_hbm, out_hbm,
             srcids_v, dstids_v, zrows_v, rows_v,
             normsrc_v, normdst_v, agg_sh, *sems):
    sub = lax.axis_index("s")
    rowbase = sub * RPT

    pltpu.sync_copy(src2d_hbm.at[pl.ds(sub * NCH, NCH)], srcids_v)
    pltpu.sync_copy(dst2d_hbm.at[pl.ds(sub * NCH, NCH)], dstids_v)
    pltpu.sync_copy(norm_hbm.at[pl.ds(rowbase, RPT)], normsrc_v)
    pltpu.sync_copy(norm_hbm.at[pl.ds(DST_OFF + rowbase, RPT)], normdst_v)

    zeros16 = jnp.zeros((16,), _f32)

    pltpu.sync_copy(z1_hbm.at[pl.ds(rowbase, RPT)], zrows_v)

    def _scale_rows(norm_v):
        def _scale_grp(g, _):
            nv = norm_v[pl.ds(g * 16, 16)]
            for u in range(16):
                s = nv[u]
                r = g * 16 + u
                zrows_v[r, pl.ds(0, 16)] = zrows_v[r, pl.ds(0, 16)] * s
                zrows_v[r, pl.ds(16, 16)] = zrows_v[r, pl.ds(16, 16)] * s
            return _

        lax.fori_loop(0, RPT // 16, _scale_grp, None)

    _scale_rows(normsrc_v)
    pltpu.sync_copy(zrows_v, scaled_hbm.at[pl.ds(rowbase, RPT)])

    def _zero_rows(r, _):
        zrows_v[r, pl.ds(0, 16)] = zeros16
        zrows_v[r, pl.ds(16, 16)] = zeros16
        return _

    lax.fori_loop(0, RPT, _zero_rows, None)
    pltpu.sync_copy(zrows_v, agg_sh.at[pl.ds(rowbase, RPT)])
    plsc.subcore_barrier()

    def _gather_wait(i, b):
        pltpu.make_async_copy(
            scaled_hbm.at[srcids_v.at[i]], rows_v.at[b], sems[b]).wait()

    def _scatter_wait(i, b):
        pltpu.make_async_copy(
            rows_v.at[b], agg_sh.at[dstids_v.at[i]], sems[NBUF + b]).wait()

    for b in range(NBUF):
        pltpu.async_copy(scaled_hbm.at[srcids_v.at[b]], rows_v.at[b], sems[b])

    def _group(g, _):
        for b in range(NBUF):
            i = g * NBUF + b
            _gather_wait(i, b)
            pltpu.async_copy(
                rows_v.at[b], agg_sh.at[dstids_v.at[i]], sems[NBUF + b],
                add=True)

            @pl.when(g < NCH // NBUF - 1)
            def _():
                _scatter_wait(i, b)
                pltpu.async_copy(
                    scaled_hbm.at[srcids_v.at[i + NBUF]], rows_v.at[b],
                    sems[b])

        return _

    lax.fori_loop(0, NCH // NBUF, _group, None)
    for b in range(NBUF):
        _scatter_wait(NCH - NBUF + b, b)

    plsc.subcore_barrier()

    pltpu.sync_copy(agg_sh.at[pl.ds(rowbase, RPT)], zrows_v)
    _scale_rows(normdst_v)
    pltpu.sync_copy(zrows_v, out_hbm.at[pl.ds(rowbase, RPT)])


def _sc_aggregate(z1, norm, src2d, dst2d):
    mesh = plsc.VectorSubcoreMesh(
        core_axis_name="c", subcore_axis_name="s", num_cores=1)
    call = pl.kernel(
        _sc_body,
        out_type=[
            jax.ShapeDtypeStruct((NROW, H1), _f32),
            jax.ShapeDtypeStruct((NROW, H1), _f32),
        ],
        mesh=mesh,
        scratch_types=[
            pltpu.VMEM((NCH, CH), jnp.int32),
            pltpu.VMEM((NCH, CH), jnp.int32),
            pltpu.VMEM((RPT, H1), _f32),
            pltpu.VMEM((NBUF, CH, H1), _f32),
            pltpu.VMEM((RPT,), _f32),
            pltpu.VMEM((RPT,), _f32),
            pltpu.VMEM_SHARED((NROW, H1), _f32),
        ] + [pltpu.SemaphoreType.DMA] * (2 * NBUF),
        compiler_params=pltpu.CompilerParams(
            needs_layout_passes=False, use_tc_tiling_on_sc=False
        ),
    )
    return call(z1, norm, src2d, dst2d)


def _tc_post_body(p_ref, w_ref, b_ref, out_ref):
    z = jnp.dot(p_ref[0:N], w_ref[...], preferred_element_type=_f32)
    out_ref[...] = jnp.maximum(z + b_ref[...], 0.0)


def _tc_post(agg, w_gc, b_gc):
    return pl.pallas_call(
        _tc_post_body,
        out_shape=jax.ShapeDtypeStruct((N, H2), _f32),
    )(agg, w_gc, b_gc)


def kernel(x, edge_index, W_lin, b_lin, W_gc, b_gc):
    src = edge_index[0]
    dst = edge_index[1]
    src2d = src.reshape(NS * NCH, CH)
    dst2d = dst.reshape(NS * NCH, CH)
    norm = _sc_norms(src, dst)
    z1 = _tc_z1(x, W_lin, b_lin.reshape(1, H1))
    _, agg = _sc_aggregate(z1, norm, src2d, dst2d)
    return _tc_post(agg, W_gc, b_gc.reshape(1, H2))
